# Initial kernel scaffold; baseline (speedup 1.0000x reference)
import jax
import jax.numpy as jnp
from jax.experimental import pallas as pl


def kernel(ori_edge_index, ori_values, pk_edge_index, pk_values, mask, drp_edge_index, drp_values, edge_embeds1, withdraw_rate, W_qkv, W_out, ln_gamma, ln_beta, ini_embeds, fnl_embeds):
    raise NotImplementedError("write your pallas kernel here")


# speedup vs baseline: 2.6757x; 2.6757x over previous
import jax
import jax.numpy as jnp
from jax import lax
from jax.experimental import pallas as pl
from jax.experimental.pallas import tpu as pltpu
from jax.experimental.pallas import tpu_sc as plsc

N = 10000
D = 256
NHEAD = 8
HDIM = 32
LB = 64
NC = 2
NS = 16
RPT = N // NS
F32 = jnp.float32
I32 = jnp.int32

E_DRP = 40000
E_PK = 120000
EP_DRP = 40960
EP_PK = 122880
SCH = 640
ACH = 128
ANCH = (EP_DRP // NS) // ACH
ZR = 125

_SC_PARAMS = pltpu.CompilerParams(use_tc_tiling_on_sc=False,
                                  needs_layout_passes=False)


def _mesh():
    return plsc.VectorSubcoreMesh(core_axis_name="c", subcore_axis_name="s",
                                  num_cores=NC, num_subcores=NS)


def _splat(i):
    return jnp.full((16,), 0, I32) + i


def _make_spmm2(nb, ep, ch, interpret=False):
    nb2 = nb // 2
    per_tile = ep // NS
    nch = per_tile // ch

    def body(row_h, col_h, val_h, x_h, y1_h, y2_h,
             idxr, idxc, vals, rows, zbuf, acc, sem):
        c = lax.axis_index("c")
        s = lax.axis_index("s")
        pltpu.sync_copy(row_h.at[s], idxr)
        pltpu.sync_copy(col_h.at[s], idxc)
        pltpu.sync_copy(val_h.at[s], vals)

        @pl.loop(0, ZR)
        def _(i):
            for j in range(LB // 16):
                zbuf[i, pl.ds(j * 16, 16)] = jnp.zeros((16,), F32)

        def layer(src_h, dst_h):
            for b in range(nb2):
                blk = c * nb2 + b
                for k in range(RPT // ZR):
                    pltpu.sync_copy(zbuf, acc.at[pl.ds(s * RPT + k * ZR, ZR)])
                plsc.subcore_barrier()

                @pl.loop(0, nch)
                def _(t):
                    pltpu.async_copy(src_h.at[blk].at[idxc.at[t]], rows,
                                     sem).wait()

                    @pl.loop(0, ch)
                    def _(i):
                        v = plsc.load_gather(vals.at[t], [_splat(i)])
                        for j in range(LB // 16):
                            sl = pl.ds(j * 16, 16)
                            rows[i, sl] = rows[i, sl] * v

                    pltpu.sync_copy(rows, acc.at[idxr.at[t]], add=True)

                plsc.subcore_barrier()
                pltpu.sync_copy(acc.at[pl.ds(s * RPT, RPT)],
                                dst_h.at[blk, pl.ds(s * RPT, RPT)])
                plsc.subcore_barrier()

        layer(x_h, y1_h)
        layer(y1_h, y2_h)

    out = (jax.ShapeDtypeStruct((nb, N, LB), F32),
           jax.ShapeDtypeStruct((nb, N, LB), F32))
    return pl.kernel(
        body,
        out_type=out,
        mesh=_mesh(),
        interpret=interpret,
        compiler_params=_SC_PARAMS,
        scratch_types=[
            pltpu.VMEM((nch, ch), I32),
            pltpu.VMEM((nch, ch), I32),
            pltpu.VMEM((nch, ch), F32),
            pltpu.VMEM((ch, LB), F32),
            pltpu.VMEM((ZR, LB), F32),
            pltpu.VMEM_SHARED((N, LB), F32),
            pltpu.SemaphoreType.DMA,
        ],
    )


def _make_attn(interpret=False):
    per_tile = EP_DRP // NS
    scale = 1.0 / (HDIM ** 0.5)

    def body(row_h, col_h, qb_h, kb_h, vb_h, agg_h, rcp_h,
             idxr, idxc, rowsA, rowsB, vbuf, exps, contrib, rcp, zb16,
             denom, aggacc, sem):
        c = lax.axis_index("c")
        s = lax.axis_index("s")
        pltpu.sync_copy(row_h.at[s], idxr)
        pltpu.sync_copy(col_h.at[s], idxc)

        @pl.loop(0, RPT)
        def _(i):
            zb16[i, pl.ds(0, 16)] = jnp.zeros((16,), F32)

        @pl.loop(0, ACH)
        def _(i):
            contrib[i, pl.ds(0, 16)] = jnp.zeros((16,), F32)

        pltpu.sync_copy(zb16, denom.at[pl.ds(s * RPT, RPT)])
        plsc.subcore_barrier()

        lanes0 = lax.iota(I32, 16)

        @pl.loop(0, ANCH)
        def _(t):
            pltpu.async_copy(qb_h.at[c].at[idxr.at[t]], rowsA, sem).wait()
            pltpu.async_copy(kb_h.at[c].at[idxc.at[t]], rowsB, sem).wait()

            @pl.loop(0, ACH // 16)
            def _(g):
                lanes = g * 16 + lanes0
                init = (jnp.zeros((16,), F32),) * 4

                @pl.loop(0, HDIM, init_carry=init)
                def accs(d, carry):
                    outs = []
                    for h in range(4):
                        colv = jnp.full((16,), h * HDIM, I32) + d
                        q = plsc.load_gather(rowsA, [lanes, colv])
                        k = plsc.load_gather(rowsB, [lanes, colv])
                        outs.append(carry[h] + q * k)
                    return tuple(outs)

                ge = s * per_tile + t * ACH + lanes
                valid = ge < E_DRP
                for h in range(4):
                    a = accs[h] * scale
                    a = jnp.where(a >= 0, a, 0.2 * a)
                    a = jnp.clip(a, -20.0, 20.0)
                    e = jnp.where(valid, jnp.exp(a), 0.0)
                    hv = jnp.full((16,), h, I32)
                    plsc.store_scatter(contrib, [lanes, hv], e)
                    plsc.store_scatter(exps, [t * ACH + lanes, hv], e)

            pltpu.sync_copy(contrib, denom.at[idxr.at[t]], add=True)

        plsc.subcore_barrier()

        pltpu.sync_copy(denom.at[pl.ds(s * RPT, RPT)], zb16)

        @pl.loop(0, RPT)
        def _(i):
            v = zb16[i, pl.ds(0, 16)]
            zb16[i, pl.ds(0, 16)] = 1.0 / (v + 1e-10)

        pltpu.sync_copy(zb16, rcp_h.at[c, pl.ds(s * RPT, RPT)])
        plsc.subcore_barrier()

        for j in range(2):
            vblk = 2 * c + j

            @pl.loop(0, ACH)
            def _(i):
                for j2 in range(LB // 16):
                    vbuf[i, pl.ds(j2 * 16, 16)] = jnp.zeros((16,), F32)

            for k in range(RPT // ACH):
                pltpu.sync_copy(vbuf, aggacc.at[pl.ds(s * RPT + k * ACH, ACH)])
            if RPT % ACH:
                pltpu.sync_copy(
                    vbuf.at[pl.ds(0, RPT % ACH)],
                    aggacc.at[pl.ds(s * RPT + (RPT // ACH) * ACH, RPT % ACH)])
            plsc.subcore_barrier()

            @pl.loop(0, ANCH)
            def _(t):
                pltpu.async_copy(vb_h.at[vblk].at[idxc.at[t]], vbuf,
                                 sem).wait()
                pltpu.async_copy(rcp_h.at[c].at[idxr.at[t]], rcp, sem).wait()

                @pl.loop(0, ACH)
                def _(i):
                    le = _splat(t * ACH + i)
                    iv = _splat(i)
                    for j2 in range(LB // 16):
                        hv = jnp.full((16,), 2 * j + j2 // 2, I32)
                        w = (plsc.load_gather(exps, [le, hv]) *
                             plsc.load_gather(rcp, [iv, hv]))
                        sl = pl.ds(j2 * 16, 16)
                        vbuf[i, sl] = vbuf[i, sl] * w

                pltpu.sync_copy(vbuf, aggacc.at[idxr.at[t]], add=True)

            plsc.subcore_barrier()
            pltpu.sync_copy(aggacc.at[pl.ds(s * RPT, RPT)],
                            agg_h.at[vblk, pl.ds(s * RPT, RPT)])
            plsc.subcore_barrier()

    out = (jax.ShapeDtypeStruct((2 * NC, N, LB), F32),
           jax.ShapeDtypeStruct((NC, N, 16), F32))
    return pl.kernel(
        body,
        out_type=out,
        mesh=_mesh(),
        interpret=interpret,
        compiler_params=_SC_PARAMS,
        scratch_types=[
            pltpu.VMEM((ANCH, ACH), I32),
            pltpu.VMEM((ANCH, ACH), I32),
            pltpu.VMEM((ACH, 2 * LB), F32),
            pltpu.VMEM((ACH, 2 * LB), F32),
            pltpu.VMEM((ACH, LB), F32),
            pltpu.VMEM((per_tile, 4), F32),
            pltpu.VMEM((ACH, 16), F32),
            pltpu.VMEM((ACH, 16), F32),
            pltpu.VMEM((RPT, 16), F32),
            pltpu.VMEM_SHARED((N, 16), F32),
            pltpu.VMEM_SHARED((N, LB), F32),
            pltpu.SemaphoreType.DMA,
        ],
    )


ROWB = 1000
G = N // ROWB


def _row_spec(width):
    return pl.BlockSpec((ROWB, width), lambda i: (i, 0))


def _full_spec(h, w):
    return pl.BlockSpec((h, w), lambda i: (0, 0))


def _wd_body(fnl, wr, o):
    o[...] = fnl[...] * wr[...]


_wd = pl.pallas_call(
    _wd_body,
    grid=(G,),
    in_specs=[_row_spec(D), pl.BlockSpec((ROWB, 1), lambda i: (i, 0))],
    out_specs=_row_spec(D),
    out_shape=jax.ShapeDtypeStruct((N, D), F32),
)


def _add3_body(a, b, c, o):
    o[...] = a[...] + b[...] + c[...]


_add3 = pl.pallas_call(
    _add3_body,
    grid=(G,),
    in_specs=[_row_spec(D)] * 3,
    out_specs=_row_spec(D),
    out_shape=jax.ShapeDtypeStruct((N, D), F32),
)


def _qkv_body(wd2, eo, w, qkv, delta):
    d = eo[...] - 0.1 * wd2[...]
    delta[...] = d
    qkv[...] = jnp.dot(d, w[...], preferred_element_type=F32)


_qkv = pl.pallas_call(
    _qkv_body,
    grid=(G,),
    in_specs=[_row_spec(D), _row_spec(D), _full_spec(D, 3 * D)],
    out_specs=[_row_spec(3 * D), _row_spec(D)],
    out_shape=(jax.ShapeDtypeStruct((N, 3 * D), F32),
               jax.ShapeDtypeStruct((N, D), F32)),
)


def _fin_body(agg, wout, delta, g, bta, fnl, h2, eb, o):
    ao = jnp.dot(agg[...], wout[...], preferred_element_type=F32)
    x = delta[...] + ao
    mu = jnp.mean(x, axis=-1, keepdims=True)
    var = jnp.mean((x - mu) ** 2, axis=-1, keepdims=True)
    x = (x - mu) / jnp.sqrt(var + 1e-5) * g[...] + bta[...]
    x = jnp.where(x >= 0, x, 0.1 * x)
    x = jnp.where(x >= 0, x, 0.1 * x)
    o[...] = fnl[...] + x + h2[...] - eb[...]


_fin = pl.pallas_call(
    _fin_body,
    grid=(G,),
    in_specs=[_row_spec(D), _full_spec(D, D), _row_spec(D),
              _full_spec(1, D), _full_spec(1, D),
              _row_spec(D), _row_spec(D), _row_spec(D)],
    out_specs=_row_spec(D),
    out_shape=jax.ShapeDtypeStruct((N, D), F32),
)


_spmm_drp = _make_spmm2(12, EP_DRP, SCH)
_spmm_pk = _make_spmm2(8, EP_PK, SCH)
_attn = _make_attn()


def _prep_edges(ei, vals, ep, ch):
    e = ei.shape[1]
    pad = ep - e
    row = jnp.concatenate([ei[0], jnp.zeros((pad,), I32)]).reshape(NS, -1, ch)
    col = jnp.concatenate([ei[1], jnp.zeros((pad,), I32)]).reshape(NS, -1, ch)
    v = jnp.concatenate([vals, jnp.zeros((pad,), F32)]).reshape(NS, -1, ch)
    return row, col, v


def _blocks4(x):
    return (x[:, :LB], x[:, LB:2 * LB], x[:, 2 * LB:3 * LB], x[:, 3 * LB:])


def _cat4(blocks):
    return jnp.concatenate(blocks, -1)


def kernel(ori_edge_index, ori_values, pk_edge_index, pk_values, mask,
           drp_edge_index, drp_values, edge_embeds1, withdraw_rate,
           W_qkv, W_out, ln_gamma, ln_beta, ini_embeds, fnl_embeds):
    drow, dcol, dval = _prep_edges(drp_edge_index, drp_values, EP_DRP, SCH)
    prow, pcol, pval = _prep_edges(pk_edge_index, pk_values, EP_PK, SCH)
    arow = drow.reshape(NS, ANCH, ACH)
    acol = dcol.reshape(NS, ANCH, ACH)

    wd0 = _wd(fnl_embeds, withdraw_rate)
    x0 = jnp.stack([*_blocks4(edge_embeds1), *_blocks4(wd0),
                    *_blocks4(ini_embeds)])
    y1, y2 = _spmm_drp(drow, dcol, dval, x0)
    lat1 = _cat4([y1[i] for i in range(4)])
    lat2 = _cat4([y2[i] for i in range(4)])
    wd2 = _cat4([y2[i] for i in range(4, 8)])
    h2 = _cat4([y2[i] for i in range(8, 12)])

    edge_embed = _add3(edge_embeds1, lat1, lat2)
    z0 = jnp.stack([*_blocks4(edge_embed), *_blocks4(ini_embeds)])
    _, z2 = _spmm_pk(prow, pcol, pval, z0)
    edge_out = _cat4([z2[i] for i in range(4)])
    e_bar = _cat4([z2[i] for i in range(4, 8)])

    qkv, delta = _qkv(wd2, edge_out, W_qkv)
    qb = jnp.stack([qkv[:, :128], qkv[:, 128:256]])
    kb = jnp.stack([qkv[:, 256:384], qkv[:, 384:512]])
    vb = jnp.stack([qkv[:, 512 + i * LB:512 + (i + 1) * LB] for i in range(4)])
    agg, _ = _attn(arow, acol, qb, kb, vb)
    agg_cat = _cat4([agg[i] for i in range(4)])

    return _fin(agg_cat, W_out, delta, ln_gamma.reshape(1, D),
                ln_beta.reshape(1, D), fnl_embeds, h2, e_bar)


# speedup vs baseline: 2.7288x; 1.0198x over previous
import jax
import jax.numpy as jnp
from jax import lax
from jax.experimental import pallas as pl
from jax.experimental.pallas import tpu as pltpu
from jax.experimental.pallas import tpu_sc as plsc

N = 10000
D = 256
NHEAD = 8
HDIM = 32
LB = 64
NC = 2
NS = 16
RPT = N // NS
F32 = jnp.float32
I32 = jnp.int32

E_DRP = 40000
E_PK = 120000
EP_DRP = 40960
EP_PK = 122880
SCH = 640
ACH = 128
ANCH = (EP_DRP // NS) // ACH
ZR = 125

_SC_PARAMS = pltpu.CompilerParams(use_tc_tiling_on_sc=False,
                                  needs_layout_passes=False)


def _mesh():
    return plsc.VectorSubcoreMesh(core_axis_name="c", subcore_axis_name="s",
                                  num_cores=NC, num_subcores=NS)


def _splat(i):
    return jnp.full((16,), 0, I32) + i


def _make_spmm2(nb, ep, ch, interpret=False):
    nb2 = nb // 2
    per_tile = ep // NS
    nch = per_tile // ch

    def body(row_h, col_h, val_h, x_h, y1_h, y2_h,
             idxr, idxc, vals, rows, zbuf, acc, sem):
        c = lax.axis_index("c")
        s = lax.axis_index("s")
        pltpu.sync_copy(row_h.at[s], idxr)
        pltpu.sync_copy(col_h.at[s], idxc)
        pltpu.sync_copy(val_h.at[s], vals)

        @pl.loop(0, ZR)
        def _(i):
            for j in range(LB // 16):
                zbuf[i, pl.ds(j * 16, 16)] = jnp.zeros((16,), F32)

        def layer(src_h, dst_h):
            for b in range(nb2):
                blk = c * nb2 + b
                for k in range(RPT // ZR):
                    pltpu.sync_copy(zbuf, acc.at[pl.ds(s * RPT + k * ZR, ZR)])
                plsc.subcore_barrier()

                @pl.loop(0, nch)
                def _(t):
                    pltpu.async_copy(src_h.at[blk].at[idxc.at[t]], rows,
                                     sem).wait()

                    @pl.loop(0, ch, unroll=8)
                    def _(i):
                        v = plsc.load_gather(vals.at[t], [_splat(i)])
                        for j in range(LB // 16):
                            sl = pl.ds(j * 16, 16)
                            rows[i, sl] = rows[i, sl] * v

                    pltpu.sync_copy(rows, acc.at[idxr.at[t]], add=True)

                plsc.subcore_barrier()
                pltpu.sync_copy(acc.at[pl.ds(s * RPT, RPT)],
                                dst_h.at[blk, pl.ds(s * RPT, RPT)])
                plsc.subcore_barrier()

        layer(x_h, y1_h)
        layer(y1_h, y2_h)

    out = (jax.ShapeDtypeStruct((nb, N, LB), F32),
           jax.ShapeDtypeStruct((nb, N, LB), F32))
    return pl.kernel(
        body,
        out_type=out,
        mesh=_mesh(),
        interpret=interpret,
        compiler_params=_SC_PARAMS,
        scratch_types=[
            pltpu.VMEM((nch, ch), I32),
            pltpu.VMEM((nch, ch), I32),
            pltpu.VMEM((nch, ch), F32),
            pltpu.VMEM((ch, LB), F32),
            pltpu.VMEM((ZR, LB), F32),
            pltpu.VMEM_SHARED((N, LB), F32),
            pltpu.SemaphoreType.DMA,
        ],
    )


def _make_attn(interpret=False):
    per_tile = EP_DRP // NS
    scale = 1.0 / (HDIM ** 0.5)

    def body(row_h, col_h, qb_h, kb_h, vb_h, agg_h, rcp_h,
             idxr, idxc, rowsA, rowsB, vbuf, exps, contrib, rcp, zb16,
             denom, aggacc, sem):
        c = lax.axis_index("c")
        s = lax.axis_index("s")
        pltpu.sync_copy(row_h.at[s], idxr)
        pltpu.sync_copy(col_h.at[s], idxc)

        @pl.loop(0, RPT)
        def _(i):
            zb16[i, pl.ds(0, 16)] = jnp.zeros((16,), F32)

        @pl.loop(0, ACH)
        def _(i):
            contrib[i, pl.ds(0, 16)] = jnp.zeros((16,), F32)

        pltpu.sync_copy(zb16, denom.at[pl.ds(s * RPT, RPT)])
        plsc.subcore_barrier()

        lanes0 = lax.iota(I32, 16)

        @pl.loop(0, ANCH)
        def _(t):
            pltpu.async_copy(qb_h.at[c].at[idxr.at[t]], rowsA, sem).wait()
            pltpu.async_copy(kb_h.at[c].at[idxc.at[t]], rowsB, sem).wait()

            @pl.loop(0, ACH // 16)
            def _(g):
                lanes = g * 16 + lanes0
                init = (jnp.zeros((16,), F32),) * 4

                @pl.loop(0, HDIM, init_carry=init, unroll=4)
                def accs(d, carry):
                    outs = []
                    for h in range(4):
                        colv = jnp.full((16,), h * HDIM, I32) + d
                        q = plsc.load_gather(rowsA, [lanes, colv])
                        k = plsc.load_gather(rowsB, [lanes, colv])
                        outs.append(carry[h] + q * k)
                    return tuple(outs)

                ge = s * per_tile + t * ACH + lanes
                valid = ge < E_DRP
                for h in range(4):
                    a = accs[h] * scale
                    a = jnp.where(a >= 0, a, 0.2 * a)
                    a = jnp.clip(a, -20.0, 20.0)
                    e = jnp.where(valid, jnp.exp(a), 0.0)
                    hv = jnp.full((16,), h, I32)
                    plsc.store_scatter(contrib, [lanes, hv], e)
                    plsc.store_scatter(exps, [t * ACH + lanes, hv], e)

            pltpu.sync_copy(contrib, denom.at[idxr.at[t]], add=True)

        plsc.subcore_barrier()

        pltpu.sync_copy(denom.at[pl.ds(s * RPT, RPT)], zb16)

        @pl.loop(0, RPT)
        def _(i):
            v = zb16[i, pl.ds(0, 16)]
            zb16[i, pl.ds(0, 16)] = 1.0 / (v + 1e-10)

        pltpu.sync_copy(zb16, rcp_h.at[c, pl.ds(s * RPT, RPT)])
        plsc.subcore_barrier()

        for j in range(2):
            vblk = 2 * c + j

            @pl.loop(0, ACH)
            def _(i):
                for j2 in range(LB // 16):
                    vbuf[i, pl.ds(j2 * 16, 16)] = jnp.zeros((16,), F32)

            for k in range(RPT // ACH):
                pltpu.sync_copy(vbuf, aggacc.at[pl.ds(s * RPT + k * ACH, ACH)])
            if RPT % ACH:
                pltpu.sync_copy(
                    vbuf.at[pl.ds(0, RPT % ACH)],
                    aggacc.at[pl.ds(s * RPT + (RPT // ACH) * ACH, RPT % ACH)])
            plsc.subcore_barrier()

            @pl.loop(0, ANCH)
            def _(t):
                pltpu.async_copy(vb_h.at[vblk].at[idxc.at[t]], vbuf,
                                 sem).wait()
                pltpu.async_copy(rcp_h.at[c].at[idxr.at[t]], rcp, sem).wait()

                @pl.loop(0, ACH, unroll=4)
                def _(i):
                    le = _splat(t * ACH + i)
                    iv = _splat(i)
                    for j2 in range(LB // 16):
                        hv = jnp.full((16,), 2 * j + j2 // 2, I32)
                        w = (plsc.load_gather(exps, [le, hv]) *
                             plsc.load_gather(rcp, [iv, hv]))
                        sl = pl.ds(j2 * 16, 16)
                        vbuf[i, sl] = vbuf[i, sl] * w

                pltpu.sync_copy(vbuf, aggacc.at[idxr.at[t]], add=True)

            plsc.subcore_barrier()
            pltpu.sync_copy(aggacc.at[pl.ds(s * RPT, RPT)],
                            agg_h.at[vblk, pl.ds(s * RPT, RPT)])
            plsc.subcore_barrier()

    out = (jax.ShapeDtypeStruct((2 * NC, N, LB), F32),
           jax.ShapeDtypeStruct((NC, N, 16), F32))
    return pl.kernel(
        body,
        out_type=out,
        mesh=_mesh(),
        interpret=interpret,
        compiler_params=_SC_PARAMS,
        scratch_types=[
            pltpu.VMEM((ANCH, ACH), I32),
            pltpu.VMEM((ANCH, ACH), I32),
            pltpu.VMEM((ACH, 2 * LB), F32),
            pltpu.VMEM((ACH, 2 * LB), F32),
            pltpu.VMEM((ACH, LB), F32),
            pltpu.VMEM((per_tile, 4), F32),
            pltpu.VMEM((ACH, 16), F32),
            pltpu.VMEM((ACH, 16), F32),
            pltpu.VMEM((RPT, 16), F32),
            pltpu.VMEM_SHARED((N, 16), F32),
            pltpu.VMEM_SHARED((N, LB), F32),
            pltpu.SemaphoreType.DMA,
        ],
    )


ROWB = 1000
G = N // ROWB


def _row_spec(width):
    return pl.BlockSpec((ROWB, width), lambda i: (i, 0))


def _full_spec(h, w):
    return pl.BlockSpec((h, w), lambda i: (0, 0))


def _wd_body(fnl, wr, o):
    o[...] = fnl[...] * wr[...]


_wd = pl.pallas_call(
    _wd_body,
    grid=(G,),
    in_specs=[_row_spec(D), pl.BlockSpec((ROWB, 1), lambda i: (i, 0))],
    out_specs=_row_spec(D),
    out_shape=jax.ShapeDtypeStruct((N, D), F32),
)


def _add3_body(a, b, c, o):
    o[...] = a[...] + b[...] + c[...]


_add3 = pl.pallas_call(
    _add3_body,
    grid=(G,),
    in_specs=[_row_spec(D)] * 3,
    out_specs=_row_spec(D),
    out_shape=jax.ShapeDtypeStruct((N, D), F32),
)


def _qkv_body(wd2, eo, w, qkv, delta):
    d = eo[...] - 0.1 * wd2[...]
    delta[...] = d
    qkv[...] = jnp.dot(d, w[...], preferred_element_type=F32)


_qkv = pl.pallas_call(
    _qkv_body,
    grid=(G,),
    in_specs=[_row_spec(D), _row_spec(D), _full_spec(D, 3 * D)],
    out_specs=[_row_spec(3 * D), _row_spec(D)],
    out_shape=(jax.ShapeDtypeStruct((N, 3 * D), F32),
               jax.ShapeDtypeStruct((N, D), F32)),
)


def _fin_body(agg, wout, delta, g, bta, fnl, h2, eb, o):
    ao = jnp.dot(agg[...], wout[...], preferred_element_type=F32)
    x = delta[...] + ao
    mu = jnp.mean(x, axis=-1, keepdims=True)
    var = jnp.mean((x - mu) ** 2, axis=-1, keepdims=True)
    x = (x - mu) / jnp.sqrt(var + 1e-5) * g[...] + bta[...]
    x = jnp.where(x >= 0, x, 0.1 * x)
    x = jnp.where(x >= 0, x, 0.1 * x)
    o[...] = fnl[...] + x + h2[...] - eb[...]


_fin = pl.pallas_call(
    _fin_body,
    grid=(G,),
    in_specs=[_row_spec(D), _full_spec(D, D), _row_spec(D),
              _full_spec(1, D), _full_spec(1, D),
              _row_spec(D), _row_spec(D), _row_spec(D)],
    out_specs=_row_spec(D),
    out_shape=jax.ShapeDtypeStruct((N, D), F32),
)


_spmm_drp = _make_spmm2(12, EP_DRP, SCH)
_spmm_pk = _make_spmm2(8, EP_PK, SCH)
_attn = _make_attn()


def _prep_edges(ei, vals, ep, ch):
    e = ei.shape[1]
    pad = ep - e
    row = jnp.concatenate([ei[0], jnp.zeros((pad,), I32)]).reshape(NS, -1, ch)
    col = jnp.concatenate([ei[1], jnp.zeros((pad,), I32)]).reshape(NS, -1, ch)
    v = jnp.concatenate([vals, jnp.zeros((pad,), F32)]).reshape(NS, -1, ch)
    return row, col, v


def _blocks4(x):
    return (x[:, :LB], x[:, LB:2 * LB], x[:, 2 * LB:3 * LB], x[:, 3 * LB:])


def _cat4(blocks):
    return jnp.concatenate(blocks, -1)


def kernel(ori_edge_index, ori_values, pk_edge_index, pk_values, mask,
           drp_edge_index, drp_values, edge_embeds1, withdraw_rate,
           W_qkv, W_out, ln_gamma, ln_beta, ini_embeds, fnl_embeds):
    drow, dcol, dval = _prep_edges(drp_edge_index, drp_values, EP_DRP, SCH)
    prow, pcol, pval = _prep_edges(pk_edge_index, pk_values, EP_PK, SCH)
    arow = drow.reshape(NS, ANCH, ACH)
    acol = dcol.reshape(NS, ANCH, ACH)

    wd0 = _wd(fnl_embeds, withdraw_rate)
    x0 = jnp.stack([*_blocks4(edge_embeds1), *_blocks4(wd0),
                    *_blocks4(ini_embeds)])
    y1, y2 = _spmm_drp(drow, dcol, dval, x0)
    lat1 = _cat4([y1[i] for i in range(4)])
    lat2 = _cat4([y2[i] for i in range(4)])
    wd2 = _cat4([y2[i] for i in range(4, 8)])
    h2 = _cat4([y2[i] for i in range(8, 12)])

    edge_embed = _add3(edge_embeds1, lat1, lat2)
    z0 = jnp.stack([*_blocks4(edge_embed), *_blocks4(ini_embeds)])
    _, z2 = _spmm_pk(prow, pcol, pval, z0)
    edge_out = _cat4([z2[i] for i in range(4)])
    e_bar = _cat4([z2[i] for i in range(4, 8)])

    qkv, delta = _qkv(wd2, edge_out, W_qkv)
    qb = jnp.stack([qkv[:, :128], qkv[:, 128:256]])
    kb = jnp.stack([qkv[:, 256:384], qkv[:, 384:512]])
    vb = jnp.stack([qkv[:, 512 + i * LB:512 + (i + 1) * LB] for i in range(4)])
    agg, _ = _attn(arow, acol, qb, kb, vb)
    agg_cat = _cat4([agg[i] for i in range(4)])

    return _fin(agg_cat, W_out, delta, ln_gamma.reshape(1, D),
                ln_beta.reshape(1, D), fnl_embeds, h2, e_bar)


# speedup vs baseline: 3.1651x; 1.1599x over previous
import jax
import jax.numpy as jnp
from jax import lax
from jax.experimental import pallas as pl
from jax.experimental.pallas import tpu as pltpu
from jax.experimental.pallas import tpu_sc as plsc

N = 10000
D = 256
NHEAD = 8
HDIM = 32
LB = 64
NC = 2
NS = 16
RPT = N // NS
F32 = jnp.float32
I32 = jnp.int32

E_DRP = 40000
E_PK = 120000
EP_DRP = 40960
EP_PK = 122880
SCH = 320
ACH = 128
ANCH = (EP_DRP // NS) // ACH
ZR = 125

_SC_PARAMS = pltpu.CompilerParams(use_tc_tiling_on_sc=False,
                                  needs_layout_passes=False)


def _mesh():
    return plsc.VectorSubcoreMesh(core_axis_name="c", subcore_axis_name="s",
                                  num_cores=NC, num_subcores=NS)


def _splat(i):
    return jnp.full((16,), 0, I32) + i


def _make_spmm2(nb, ep, ch, interpret=False):
    nb2 = nb // 2
    per_tile = ep // NS
    nch = per_tile // ch

    def body(row_h, col_h, val_h, x_h, y1_h, y2_h,
             idxr, idxc, vals, rows0, rows1, zbuf, acc,
             gsem0, gsem1, ssem0, ssem1):
        c = lax.axis_index("c")
        s = lax.axis_index("s")
        rows = (rows0, rows1)
        gsem = (gsem0, gsem1)
        ssem = (ssem0, ssem1)
        pltpu.sync_copy(row_h.at[s], idxr)
        pltpu.sync_copy(col_h.at[s], idxc)
        pltpu.sync_copy(val_h.at[s], vals)

        @pl.loop(0, ZR)
        def _(i):
            for j in range(LB // 16):
                zbuf[i, pl.ds(j * 16, 16)] = jnp.zeros((16,), F32)

        def layer(src_h, dst_h):
            @pl.loop(0, nb2)
            def _(b):
                blk = c * nb2 + b

                def fire_g(t, p):
                    pltpu.async_copy(src_h.at[blk].at[idxc.at[t]], rows[p],
                                     gsem[p])

                def wait_g(t, p):
                    pltpu.make_async_copy(src_h.at[blk].at[idxc.at[t]],
                                          rows[p], gsem[p]).wait()

                def fire_s(t, p):
                    pltpu.async_copy(rows[p], acc.at[idxr.at[t]], ssem[p],
                                     add=True)

                def wait_s(t, p):
                    pltpu.make_async_copy(rows[p], acc.at[idxr.at[t]],
                                          ssem[p]).wait()

                def scale(p, t):
                    buf = rows[p]

                    @pl.loop(0, ch, unroll=8)
                    def _(i):
                        v = plsc.load_gather(vals.at[t], [_splat(i)])
                        for j in range(LB // 16):
                            sl = pl.ds(j * 16, 16)
                            buf[i, sl] = buf[i, sl] * v

                fire_g(0, 0)
                for k in range(RPT // ZR):
                    pltpu.sync_copy(zbuf, acc.at[pl.ds(s * RPT + k * ZR, ZR)])
                plsc.subcore_barrier()
                wait_g(0, 0)
                fire_g(1, 1)
                scale(0, 0)
                fire_s(0, 0)

                @pl.loop(0, (nch - 2) // 2)
                def _(tt):
                    t = 2 * tt + 1
                    wait_g(t, 1)
                    wait_s(t - 1, 0)
                    fire_g(t + 1, 0)
                    scale(1, t)
                    fire_s(t, 1)
                    wait_g(t + 1, 0)
                    wait_s(t, 1)
                    fire_g(t + 2, 1)
                    scale(0, t + 1)
                    fire_s(t + 1, 0)

                t_last = nch - 1
                wait_g(t_last, 1)
                wait_s(t_last - 1, 0)
                scale(1, t_last)
                fire_s(t_last, 1)
                wait_s(t_last, 1)
                plsc.subcore_barrier()
                pltpu.sync_copy(acc.at[pl.ds(s * RPT, RPT)],
                                dst_h.at[blk, pl.ds(s * RPT, RPT)])
                plsc.subcore_barrier()

        layer(x_h, y1_h)
        layer(y1_h, y2_h)

    out = (jax.ShapeDtypeStruct((nb, N, LB), F32),
           jax.ShapeDtypeStruct((nb, N, LB), F32))
    return pl.kernel(
        body,
        out_type=out,
        mesh=_mesh(),
        interpret=interpret,
        compiler_params=_SC_PARAMS,
        scratch_types=[
            pltpu.VMEM((nch, ch), I32),
            pltpu.VMEM((nch, ch), I32),
            pltpu.VMEM((nch, ch), F32),
            pltpu.VMEM((ch, LB), F32),
            pltpu.VMEM((ch, LB), F32),
            pltpu.VMEM((ZR, LB), F32),
            pltpu.VMEM_SHARED((N, LB), F32),
            pltpu.SemaphoreType.DMA,
            pltpu.SemaphoreType.DMA,
            pltpu.SemaphoreType.DMA,
            pltpu.SemaphoreType.DMA,
        ],
    )


def _make_attn(interpret=False):
    per_tile = EP_DRP // NS
    scale = 1.0 / (HDIM ** 0.5)

    def body(row_h, col_h, qb_h, kb_h, vb_h, agg_h, rcp_h,
             idxr, idxc, rowsA, rowsB, vbuf, exps, contrib, rcp, zb16,
             denom, aggacc, sem):
        c = lax.axis_index("c")
        s = lax.axis_index("s")
        pltpu.sync_copy(row_h.at[s], idxr)
        pltpu.sync_copy(col_h.at[s], idxc)

        @pl.loop(0, RPT)
        def _(i):
            zb16[i, pl.ds(0, 16)] = jnp.zeros((16,), F32)

        @pl.loop(0, ACH)
        def _(i):
            contrib[i, pl.ds(0, 16)] = jnp.zeros((16,), F32)

        pltpu.sync_copy(zb16, denom.at[pl.ds(s * RPT, RPT)])
        plsc.subcore_barrier()

        lanes0 = lax.iota(I32, 16)

        @pl.loop(0, ANCH)
        def _(t):
            pltpu.async_copy(qb_h.at[c].at[idxr.at[t]], rowsA, sem).wait()
            pltpu.async_copy(kb_h.at[c].at[idxc.at[t]], rowsB, sem).wait()

            @pl.loop(0, ACH // 16)
            def _(g):
                lanes = g * 16 + lanes0
                init = (jnp.zeros((16,), F32),) * 4

                @pl.loop(0, HDIM, init_carry=init, unroll=4)
                def accs(d, carry):
                    outs = []
                    for h in range(4):
                        colv = jnp.full((16,), h * HDIM, I32) + d
                        q = plsc.load_gather(rowsA, [lanes, colv])
                        k = plsc.load_gather(rowsB, [lanes, colv])
                        outs.append(carry[h] + q * k)
                    return tuple(outs)

                ge = s * per_tile + t * ACH + lanes
                valid = ge < E_DRP
                for h in range(4):
                    a = accs[h] * scale
                    a = jnp.where(a >= 0, a, 0.2 * a)
                    a = jnp.clip(a, -20.0, 20.0)
                    e = jnp.where(valid, jnp.exp(a), 0.0)
                    hv = jnp.full((16,), h, I32)
                    plsc.store_scatter(contrib, [lanes, hv], e)
                    plsc.store_scatter(exps, [t * ACH + lanes, hv], e)

            pltpu.sync_copy(contrib, denom.at[idxr.at[t]], add=True)

        plsc.subcore_barrier()

        pltpu.sync_copy(denom.at[pl.ds(s * RPT, RPT)], zb16)

        @pl.loop(0, RPT)
        def _(i):
            v = zb16[i, pl.ds(0, 16)]
            zb16[i, pl.ds(0, 16)] = 1.0 / (v + 1e-10)

        pltpu.sync_copy(zb16, rcp_h.at[c, pl.ds(s * RPT, RPT)])
        plsc.subcore_barrier()

        for j in range(2):
            vblk = 2 * c + j

            @pl.loop(0, ACH)
            def _(i):
                for j2 in range(LB // 16):
                    vbuf[i, pl.ds(j2 * 16, 16)] = jnp.zeros((16,), F32)

            for k in range(RPT // ACH):
                pltpu.sync_copy(vbuf, aggacc.at[pl.ds(s * RPT + k * ACH, ACH)])
            if RPT % ACH:
                pltpu.sync_copy(
                    vbuf.at[pl.ds(0, RPT % ACH)],
                    aggacc.at[pl.ds(s * RPT + (RPT // ACH) * ACH, RPT % ACH)])
            plsc.subcore_barrier()

            @pl.loop(0, ANCH)
            def _(t):
                pltpu.async_copy(vb_h.at[vblk].at[idxc.at[t]], vbuf,
                                 sem).wait()
                pltpu.async_copy(rcp_h.at[c].at[idxr.at[t]], rcp, sem).wait()

                @pl.loop(0, ACH, unroll=4)
                def _(i):
                    le = _splat(t * ACH + i)
                    iv = _splat(i)
                    for j2 in range(LB // 16):
                        hv = jnp.full((16,), 2 * j + j2 // 2, I32)
                        w = (plsc.load_gather(exps, [le, hv]) *
                             plsc.load_gather(rcp, [iv, hv]))
                        sl = pl.ds(j2 * 16, 16)
                        vbuf[i, sl] = vbuf[i, sl] * w

                pltpu.sync_copy(vbuf, aggacc.at[idxr.at[t]], add=True)

            plsc.subcore_barrier()
            pltpu.sync_copy(aggacc.at[pl.ds(s * RPT, RPT)],
                            agg_h.at[vblk, pl.ds(s * RPT, RPT)])
            plsc.subcore_barrier()

    out = (jax.ShapeDtypeStruct((2 * NC, N, LB), F32),
           jax.ShapeDtypeStruct((NC, N, 16), F32))
    return pl.kernel(
        body,
        out_type=out,
        mesh=_mesh(),
        interpret=interpret,
        compiler_params=_SC_PARAMS,
        scratch_types=[
            pltpu.VMEM((ANCH, ACH), I32),
            pltpu.VMEM((ANCH, ACH), I32),
            pltpu.VMEM((ACH, 2 * LB), F32),
            pltpu.VMEM((ACH, 2 * LB), F32),
            pltpu.VMEM((ACH, LB), F32),
            pltpu.VMEM((per_tile, 4), F32),
            pltpu.VMEM((ACH, 16), F32),
            pltpu.VMEM((ACH, 16), F32),
            pltpu.VMEM((RPT, 16), F32),
            pltpu.VMEM_SHARED((N, 16), F32),
            pltpu.VMEM_SHARED((N, LB), F32),
            pltpu.SemaphoreType.DMA,
        ],
    )


ROWB = 1000
G = N // ROWB


def _row_spec(width):
    return pl.BlockSpec((ROWB, width), lambda i: (i, 0))


def _full_spec(h, w):
    return pl.BlockSpec((h, w), lambda i: (0, 0))


def _wd_body(fnl, wr, o):
    o[...] = fnl[...] * wr[...]


_wd = pl.pallas_call(
    _wd_body,
    grid=(G,),
    in_specs=[_row_spec(D), pl.BlockSpec((ROWB, 1), lambda i: (i, 0))],
    out_specs=_row_spec(D),
    out_shape=jax.ShapeDtypeStruct((N, D), F32),
)


def _add3_body(a, b, c, o):
    o[...] = a[...] + b[...] + c[...]


_add3 = pl.pallas_call(
    _add3_body,
    grid=(G,),
    in_specs=[_row_spec(D)] * 3,
    out_specs=_row_spec(D),
    out_shape=jax.ShapeDtypeStruct((N, D), F32),
)


def _qkv_body(wd2, eo, w, qkv, delta):
    d = eo[...] - 0.1 * wd2[...]
    delta[...] = d
    qkv[...] = jnp.dot(d, w[...], preferred_element_type=F32)


_qkv = pl.pallas_call(
    _qkv_body,
    grid=(G,),
    in_specs=[_row_spec(D), _row_spec(D), _full_spec(D, 3 * D)],
    out_specs=[_row_spec(3 * D), _row_spec(D)],
    out_shape=(jax.ShapeDtypeStruct((N, 3 * D), F32),
               jax.ShapeDtypeStruct((N, D), F32)),
)


def _fin_body(agg, wout, delta, g, bta, fnl, h2, eb, o):
    ao = jnp.dot(agg[...], wout[...], preferred_element_type=F32)
    x = delta[...] + ao
    mu = jnp.mean(x, axis=-1, keepdims=True)
    var = jnp.mean((x - mu) ** 2, axis=-1, keepdims=True)
    x = (x - mu) / jnp.sqrt(var + 1e-5) * g[...] + bta[...]
    x = jnp.where(x >= 0, x, 0.1 * x)
    x = jnp.where(x >= 0, x, 0.1 * x)
    o[...] = fnl[...] + x + h2[...] - eb[...]


_fin = pl.pallas_call(
    _fin_body,
    grid=(G,),
    in_specs=[_row_spec(D), _full_spec(D, D), _row_spec(D),
              _full_spec(1, D), _full_spec(1, D),
              _row_spec(D), _row_spec(D), _row_spec(D)],
    out_specs=_row_spec(D),
    out_shape=jax.ShapeDtypeStruct((N, D), F32),
)


_spmm_drp = _make_spmm2(12, EP_DRP, SCH)
_spmm_pk = _make_spmm2(8, EP_PK, SCH)
_attn = _make_attn()


def _prep_edges(ei, vals, ep, ch):
    e = ei.shape[1]
    pad = ep - e
    row = jnp.concatenate([ei[0], jnp.zeros((pad,), I32)]).reshape(NS, -1, ch)
    col = jnp.concatenate([ei[1], jnp.zeros((pad,), I32)]).reshape(NS, -1, ch)
    v = jnp.concatenate([vals, jnp.zeros((pad,), F32)]).reshape(NS, -1, ch)
    return row, col, v


def _blocks4(x):
    return (x[:, :LB], x[:, LB:2 * LB], x[:, 2 * LB:3 * LB], x[:, 3 * LB:])


def _cat4(blocks):
    return jnp.concatenate(blocks, -1)


def kernel(ori_edge_index, ori_values, pk_edge_index, pk_values, mask,
           drp_edge_index, drp_values, edge_embeds1, withdraw_rate,
           W_qkv, W_out, ln_gamma, ln_beta, ini_embeds, fnl_embeds):
    drow, dcol, dval = _prep_edges(drp_edge_index, drp_values, EP_DRP, SCH)
    prow, pcol, pval = _prep_edges(pk_edge_index, pk_values, EP_PK, SCH)
    arow = drow.reshape(NS, ANCH, ACH)
    acol = dcol.reshape(NS, ANCH, ACH)

    wd0 = _wd(fnl_embeds, withdraw_rate)
    x0 = jnp.stack([*_blocks4(edge_embeds1), *_blocks4(wd0),
                    *_blocks4(ini_embeds)])
    y1, y2 = _spmm_drp(drow, dcol, dval, x0)
    lat1 = _cat4([y1[i] for i in range(4)])
    lat2 = _cat4([y2[i] for i in range(4)])
    wd2 = _cat4([y2[i] for i in range(4, 8)])
    h2 = _cat4([y2[i] for i in range(8, 12)])

    edge_embed = _add3(edge_embeds1, lat1, lat2)
    z0 = jnp.stack([*_blocks4(edge_embed), *_blocks4(ini_embeds)])
    _, z2 = _spmm_pk(prow, pcol, pval, z0)
    edge_out = _cat4([z2[i] for i in range(4)])
    e_bar = _cat4([z2[i] for i in range(4, 8)])

    qkv, delta = _qkv(wd2, edge_out, W_qkv)
    qb = jnp.stack([qkv[:, :128], qkv[:, 128:256]])
    kb = jnp.stack([qkv[:, 256:384], qkv[:, 384:512]])
    vb = jnp.stack([qkv[:, 512 + i * LB:512 + (i + 1) * LB] for i in range(4)])
    agg, _ = _attn(arow, acol, qb, kb, vb)
    agg_cat = _cat4([agg[i] for i in range(4)])

    return _fin(agg_cat, W_out, delta, ln_gamma.reshape(1, D),
                ln_beta.reshape(1, D), fnl_embeds, h2, e_bar)


# speedup vs baseline: 4.1532x; 1.3122x over previous
import jax
import jax.numpy as jnp
from jax import lax
from jax.experimental import pallas as pl
from jax.experimental.pallas import tpu as pltpu
from jax.experimental.pallas import tpu_sc as plsc

N = 10000
D = 256
NHEAD = 8
HDIM = 32
LB = 64
NC = 2
NS = 16
RPT = N // NS
F32 = jnp.float32
I32 = jnp.int32

E_DRP = 40000
E_PK = 120000
EP_DRP = 40960
EP_PK = 122880
SCH = 320
ACH = 80
ANCH = (EP_DRP // NS) // ACH
ZB = 80

_SC_PARAMS = pltpu.CompilerParams(use_tc_tiling_on_sc=False,
                                  needs_layout_passes=False)


def _mesh():
    return plsc.VectorSubcoreMesh(core_axis_name="c", subcore_axis_name="s",
                                  num_cores=NC, num_subcores=NS)


def _splat(i):
    return jnp.full((16,), 0, I32) + i


def _make_spmm2(nb, ep, ch, interpret=False):
    nb2 = nb // 2
    per_tile = ep // NS
    nch = per_tile // ch

    def body(row_h, col_h, val_h, x_h, y1_h, y2_h,
             idxr, idxc, vals, rows0, rows1, zbuf, acc, srcs,
             gsem0, gsem1, ssem0, ssem1):
        c = lax.axis_index("c")
        s = lax.axis_index("s")
        rows = (rows0, rows1)
        gsem = (gsem0, gsem1)
        ssem = (ssem0, ssem1)
        pltpu.sync_copy(row_h.at[s], idxr)
        pltpu.sync_copy(col_h.at[s], idxc)
        pltpu.sync_copy(val_h.at[s], vals)

        @pl.loop(0, ZB)
        def _(i):
            for j in range(LB // 16):
                zbuf[i, pl.ds(j * 16, 16)] = jnp.zeros((16,), F32)

        def layer(src_h, dst_h):
            @pl.loop(0, nb2)
            def _(b):
                blk = c * nb2 + b

                def fire_g(t, p):
                    pltpu.async_copy(srcs.at[idxc.at[t]], rows[p], gsem[p])

                def wait_g(t, p):
                    pltpu.make_async_copy(srcs.at[idxc.at[t]], rows[p],
                                          gsem[p]).wait()

                def fire_s(t, p):
                    pltpu.async_copy(rows[p], acc.at[idxr.at[t]], ssem[p],
                                     add=True)

                def wait_s(t, p):
                    pltpu.make_async_copy(rows[p], acc.at[idxr.at[t]],
                                          ssem[p]).wait()

                def scale(p, t):
                    buf = rows[p]

                    @pl.loop(0, ch, unroll=8)
                    def _(i):
                        v = plsc.load_gather(vals.at[t], [_splat(i)])
                        for j in range(LB // 16):
                            sl = pl.ds(j * 16, 16)
                            buf[i, sl] = buf[i, sl] * v

                pltpu.sync_copy(src_h.at[blk, pl.ds(s * RPT, RPT)],
                                srcs.at[pl.ds(s * RPT, RPT)])
                for k in range(RPT // ZB):
                    pltpu.sync_copy(zbuf, acc.at[pl.ds(s * RPT + k * ZB, ZB)])
                if RPT % ZB:
                    pltpu.sync_copy(
                        zbuf.at[pl.ds(0, RPT % ZB)],
                        acc.at[pl.ds(s * RPT + (RPT // ZB) * ZB, RPT % ZB)])
                plsc.subcore_barrier()
                fire_g(0, 0)
                wait_g(0, 0)
                fire_g(1, 1)
                scale(0, 0)
                fire_s(0, 0)

                @pl.loop(0, (nch - 2) // 2)
                def _(tt):
                    t = 2 * tt + 1
                    wait_g(t, 1)
                    wait_s(t - 1, 0)
                    fire_g(t + 1, 0)
                    scale(1, t)
                    fire_s(t, 1)
                    wait_g(t + 1, 0)
                    wait_s(t, 1)
                    fire_g(t + 2, 1)
                    scale(0, t + 1)
                    fire_s(t + 1, 0)

                t_last = nch - 1
                wait_g(t_last, 1)
                wait_s(t_last - 1, 0)
                scale(1, t_last)
                fire_s(t_last, 1)
                wait_s(t_last, 1)
                plsc.subcore_barrier()
                pltpu.sync_copy(acc.at[pl.ds(s * RPT, RPT)],
                                dst_h.at[blk, pl.ds(s * RPT, RPT)])
                plsc.subcore_barrier()

        layer(x_h, y1_h)
        layer(y1_h, y2_h)

    out = (jax.ShapeDtypeStruct((nb, N, LB), F32),
           jax.ShapeDtypeStruct((nb, N, LB), F32))
    return pl.kernel(
        body,
        out_type=out,
        mesh=_mesh(),
        interpret=interpret,
        compiler_params=_SC_PARAMS,
        scratch_types=[
            pltpu.VMEM((nch, ch), I32),
            pltpu.VMEM((nch, ch), I32),
            pltpu.VMEM((nch, ch), F32),
            pltpu.VMEM((ch, LB), F32),
            pltpu.VMEM((ch, LB), F32),
            pltpu.VMEM((ZB, LB), F32),
            pltpu.VMEM_SHARED((N, LB), F32),
            pltpu.VMEM_SHARED((N, LB), F32),
            pltpu.SemaphoreType.DMA,
            pltpu.SemaphoreType.DMA,
            pltpu.SemaphoreType.DMA,
            pltpu.SemaphoreType.DMA,
        ],
    )


def _make_attn(interpret=False):
    per_tile = EP_DRP // NS
    scale = 1.0 / (HDIM ** 0.5)
    nch = ANCH
    NF = RPT // ACH
    NR = RPT - NF * ACH

    def body(row_h, col_h, qb_h, kb_h, vb_h, agg_h,
             idxr, idxc, rowsA0, rowsA1, rowsB0, rowsB1, vbuf0, vbuf1,
             exps, contrib, dbuf, denom, aggacc,
             sem0, sem1, sem2, sem3):
        c = lax.axis_index("c")
        s = lax.axis_index("s")
        rowsA = (rowsA0, rowsA1)
        rowsB = (rowsB0, rowsB1)
        vbuf = (vbuf0, vbuf1)
        qsem = (sem0, sem1)
        ksem = (sem2, sem3)
        pltpu.sync_copy(row_h.at[s], idxr)
        pltpu.sync_copy(col_h.at[s], idxc)

        @pl.loop(0, ACH)
        def _(i):
            contrib[i, pl.ds(0, 16)] = jnp.zeros((16,), F32)

        for k in range(NF):
            pltpu.sync_copy(contrib, denom.at[pl.ds(s * RPT + k * ACH, ACH)])
        if NR:
            pltpu.sync_copy(contrib.at[pl.ds(0, NR)],
                            denom.at[pl.ds(s * RPT + NF * ACH, NR)])
        plsc.subcore_barrier()

        lanes0 = lax.iota(I32, 16)

        def fire_qk(t, p):
            pltpu.async_copy(qb_h.at[c].at[idxr.at[t]], rowsA[p], qsem[p])
            pltpu.async_copy(kb_h.at[c].at[idxc.at[t]], rowsB[p], ksem[p])

        def wait_qk(t, p):
            pltpu.make_async_copy(qb_h.at[c].at[idxr.at[t]], rowsA[p],
                                  qsem[p]).wait()
            pltpu.make_async_copy(kb_h.at[c].at[idxc.at[t]], rowsB[p],
                                  ksem[p]).wait()

        def logits(t, p):
            bufA = rowsA[p]
            bufB = rowsB[p]

            @pl.loop(0, ACH // 16)
            def _(g):
                lanes = g * 16 + lanes0
                init = (jnp.zeros((16,), F32),) * 4

                @pl.loop(0, HDIM, init_carry=init, unroll=4)
                def accs(d, carry):
                    outs = []
                    for h in range(4):
                        colv = jnp.full((16,), h * HDIM, I32) + d
                        q = plsc.load_gather(bufA, [lanes, colv])
                        k = plsc.load_gather(bufB, [lanes, colv])
                        outs.append(carry[h] + q * k)
                    return tuple(outs)

                ge = s * per_tile + t * ACH + lanes
                valid = ge < E_DRP
                for h in range(4):
                    a = accs[h] * scale
                    a = jnp.where(a >= 0, a, 0.2 * a)
                    a = jnp.clip(a, -20.0, 20.0)
                    e = jnp.where(valid, jnp.exp(a), 0.0)
                    hv = jnp.full((16,), h, I32)
                    plsc.store_scatter(contrib, [lanes, hv], e)
                    plsc.store_scatter(exps, [t * ACH + lanes, hv], e)

            pltpu.sync_copy(contrib, denom.at[idxr.at[t]], add=True)

        fire_qk(0, 0)
        wait_qk(0, 0)
        fire_qk(1, 1)
        logits(0, 0)

        @pl.loop(0, (nch - 2) // 2)
        def _(tt):
            t = 2 * tt + 1
            wait_qk(t, 1)
            fire_qk(t + 1, 0)
            logits(t, 1)
            wait_qk(t + 1, 0)
            fire_qk(t + 2, 1)
            logits(t + 1, 0)

        wait_qk(nch - 1, 1)
        logits(nch - 1, 1)
        plsc.subcore_barrier()

        vsem = qsem
        ssem = ksem

        for j in range(2):
            vblk = 2 * c + j

            @pl.loop(0, ACH)
            def _(i):
                for j2 in range(LB // 16):
                    vbuf0[i, pl.ds(j2 * 16, 16)] = jnp.zeros((16,), F32)

            for k in range(NF):
                pltpu.sync_copy(vbuf0, aggacc.at[pl.ds(s * RPT + k * ACH, ACH)])
            if NR:
                pltpu.sync_copy(vbuf0.at[pl.ds(0, NR)],
                                aggacc.at[pl.ds(s * RPT + NF * ACH, NR)])
            plsc.subcore_barrier()

            def fire_v(t, p):
                pltpu.async_copy(vb_h.at[vblk].at[idxc.at[t]], vbuf[p],
                                 vsem[p])

            def wait_v(t, p):
                pltpu.make_async_copy(vb_h.at[vblk].at[idxc.at[t]], vbuf[p],
                                      vsem[p]).wait()

            def fire_s(t, p):
                pltpu.async_copy(vbuf[p], aggacc.at[idxr.at[t]], ssem[p],
                                 add=True)

            def wait_s(t, p):
                pltpu.make_async_copy(vbuf[p], aggacc.at[idxr.at[t]],
                                      ssem[p]).wait()

            def vscale(t, p):
                buf = vbuf[p]

                @pl.loop(0, ACH, unroll=4)
                def _(i):
                    le = _splat(t * ACH + i)
                    for j2 in range(LB // 16):
                        hv = jnp.full((16,), j2 // 2, I32)
                        w = plsc.load_gather(exps, [le, hv + (2 * j)])
                        sl = pl.ds(j2 * 16, 16)
                        buf[i, sl] = buf[i, sl] * w

            fire_v(0, 0)
            wait_v(0, 0)
            fire_v(1, 1)
            vscale(0, 0)
            fire_s(0, 0)

            @pl.loop(0, (nch - 2) // 2)
            def _(tt):
                t = 2 * tt + 1
                wait_v(t, 1)
                wait_s(t - 1, 0)
                fire_v(t + 1, 0)
                vscale(t, 1)
                fire_s(t, 1)
                wait_v(t + 1, 0)
                wait_s(t, 1)
                fire_v(t + 2, 1)
                vscale(t + 1, 0)
                fire_s(t + 1, 0)

            wait_v(nch - 1, 1)
            wait_s(nch - 2, 0)
            vscale(nch - 1, 1)
            fire_s(nch - 1, 1)
            wait_s(nch - 1, 1)
            plsc.subcore_barrier()

            for k in range(NF + (1 if NR else 0)):
                nrows = ACH if k < NF else NR
                base = s * RPT + k * ACH
                if nrows == ACH:
                    pltpu.sync_copy(aggacc.at[pl.ds(base, ACH)], vbuf0)
                    pltpu.sync_copy(denom.at[pl.ds(base, ACH)], dbuf)
                else:
                    pltpu.sync_copy(aggacc.at[pl.ds(base, nrows)],
                                    vbuf0.at[pl.ds(0, nrows)])
                    pltpu.sync_copy(denom.at[pl.ds(base, nrows)],
                                    dbuf.at[pl.ds(0, nrows)])

                @pl.loop(0, nrows, unroll=2)
                def _(i):
                    d = dbuf[i, pl.ds(0, 16)]
                    dbuf[i, pl.ds(0, 16)] = 1.0 / (d + 1e-10)
                    iv = _splat(i)
                    for j2 in range(LB // 16):
                        hv = jnp.full((16,), j2 // 2, I32)
                        r = plsc.load_gather(dbuf, [iv, hv + (2 * j)])
                        sl = pl.ds(j2 * 16, 16)
                        vbuf0[i, sl] = vbuf0[i, sl] * r

                if nrows == ACH:
                    pltpu.sync_copy(vbuf0, agg_h.at[vblk, pl.ds(base, ACH)])
                else:
                    pltpu.sync_copy(vbuf0.at[pl.ds(0, nrows)],
                                    agg_h.at[vblk, pl.ds(base, nrows)])
            plsc.subcore_barrier()

    out = jax.ShapeDtypeStruct((2 * NC, N, LB), F32)
    return pl.kernel(
        body,
        out_type=out,
        mesh=_mesh(),
        interpret=interpret,
        compiler_params=_SC_PARAMS,
        scratch_types=[
            pltpu.VMEM((ANCH, ACH), I32),
            pltpu.VMEM((ANCH, ACH), I32),
            pltpu.VMEM((ACH, 2 * LB), F32),
            pltpu.VMEM((ACH, 2 * LB), F32),
            pltpu.VMEM((ACH, 2 * LB), F32),
            pltpu.VMEM((ACH, 2 * LB), F32),
            pltpu.VMEM((ACH, LB), F32),
            pltpu.VMEM((ACH, LB), F32),
            pltpu.VMEM((per_tile, 4), F32),
            pltpu.VMEM((ACH, 16), F32),
            pltpu.VMEM((ACH, 16), F32),
            pltpu.VMEM_SHARED((N, 16), F32),
            pltpu.VMEM_SHARED((N, LB), F32),
            pltpu.SemaphoreType.DMA,
            pltpu.SemaphoreType.DMA,
            pltpu.SemaphoreType.DMA,
            pltpu.SemaphoreType.DMA,
        ],
    )


ROWB = 1000
G = N // ROWB


def _row_spec(width):
    return pl.BlockSpec((ROWB, width), lambda i: (i, 0))


def _full_spec(h, w):
    return pl.BlockSpec((h, w), lambda i: (0, 0))


def _wd_body(fnl, wr, o):
    o[...] = fnl[...] * wr[...]


_wd = pl.pallas_call(
    _wd_body,
    grid=(G,),
    in_specs=[_row_spec(D), pl.BlockSpec((ROWB, 1), lambda i: (i, 0))],
    out_specs=_row_spec(D),
    out_shape=jax.ShapeDtypeStruct((N, D), F32),
)


def _add3_body(a, b, c, o):
    o[...] = a[...] + b[...] + c[...]


_add3 = pl.pallas_call(
    _add3_body,
    grid=(G,),
    in_specs=[_row_spec(D)] * 3,
    out_specs=_row_spec(D),
    out_shape=jax.ShapeDtypeStruct((N, D), F32),
)


def _qkv_body(wd2, eo, w, qkv, delta):
    d = eo[...] - 0.1 * wd2[...]
    delta[...] = d
    qkv[...] = jnp.dot(d, w[...], preferred_element_type=F32)


_qkv = pl.pallas_call(
    _qkv_body,
    grid=(G,),
    in_specs=[_row_spec(D), _row_spec(D), _full_spec(D, 3 * D)],
    out_specs=[_row_spec(3 * D), _row_spec(D)],
    out_shape=(jax.ShapeDtypeStruct((N, 3 * D), F32),
               jax.ShapeDtypeStruct((N, D), F32)),
)


def _fin_body(agg, wout, delta, g, bta, fnl, h2, eb, o):
    ao = jnp.dot(agg[...], wout[...], preferred_element_type=F32)
    x = delta[...] + ao
    mu = jnp.mean(x, axis=-1, keepdims=True)
    var = jnp.mean((x - mu) ** 2, axis=-1, keepdims=True)
    x = (x - mu) / jnp.sqrt(var + 1e-5) * g[...] + bta[...]
    x = jnp.where(x >= 0, x, 0.1 * x)
    x = jnp.where(x >= 0, x, 0.1 * x)
    o[...] = fnl[...] + x + h2[...] - eb[...]


_fin = pl.pallas_call(
    _fin_body,
    grid=(G,),
    in_specs=[_row_spec(D), _full_spec(D, D), _row_spec(D),
              _full_spec(1, D), _full_spec(1, D),
              _row_spec(D), _row_spec(D), _row_spec(D)],
    out_specs=_row_spec(D),
    out_shape=jax.ShapeDtypeStruct((N, D), F32),
)


_spmm_drp = _make_spmm2(12, EP_DRP, 256)
_spmm_pk = _make_spmm2(8, EP_PK, 160)
_attn = _make_attn()


def _prep_edges(ei, vals, ep, ch):
    e = ei.shape[1]
    pad = ep - e
    row = jnp.concatenate([ei[0], jnp.zeros((pad,), I32)]).reshape(NS, -1, ch)
    col = jnp.concatenate([ei[1], jnp.zeros((pad,), I32)]).reshape(NS, -1, ch)
    v = jnp.concatenate([vals, jnp.zeros((pad,), F32)]).reshape(NS, -1, ch)
    return row, col, v


def _blocks4(x):
    return (x[:, :LB], x[:, LB:2 * LB], x[:, 2 * LB:3 * LB], x[:, 3 * LB:])


def _cat4(blocks):
    return jnp.concatenate(blocks, -1)


def kernel(ori_edge_index, ori_values, pk_edge_index, pk_values, mask,
           drp_edge_index, drp_values, edge_embeds1, withdraw_rate,
           W_qkv, W_out, ln_gamma, ln_beta, ini_embeds, fnl_embeds):
    drow, dcol, dval = _prep_edges(drp_edge_index, drp_values, EP_DRP, 256)
    prow, pcol, pval = _prep_edges(pk_edge_index, pk_values, EP_PK, 160)
    arow = drow.reshape(NS, ANCH, ACH)
    acol = dcol.reshape(NS, ANCH, ACH)

    wd0 = _wd(fnl_embeds, withdraw_rate)
    x0 = jnp.stack([*_blocks4(edge_embeds1), *_blocks4(wd0),
                    *_blocks4(ini_embeds)])
    y1, y2 = _spmm_drp(drow, dcol, dval, x0)
    lat1 = _cat4([y1[i] for i in range(4)])
    lat2 = _cat4([y2[i] for i in range(4)])
    wd2 = _cat4([y2[i] for i in range(4, 8)])
    h2 = _cat4([y2[i] for i in range(8, 12)])

    edge_embed = _add3(edge_embeds1, lat1, lat2)
    z0 = jnp.stack([*_blocks4(edge_embed), *_blocks4(ini_embeds)])
    _, z2 = _spmm_pk(prow, pcol, pval, z0)
    edge_out = _cat4([z2[i] for i in range(4)])
    e_bar = _cat4([z2[i] for i in range(4, 8)])

    qkv, delta = _qkv(wd2, edge_out, W_qkv)
    qb = jnp.stack([qkv[:, :128], qkv[:, 128:256]])
    kb = jnp.stack([qkv[:, 256:384], qkv[:, 384:512]])
    vb = jnp.stack([qkv[:, 512 + i * LB:512 + (i + 1) * LB] for i in range(4)])
    agg = _attn(arow, acol, qb, kb, vb)
    agg_cat = _cat4([agg[i] for i in range(4)])

    return _fin(agg_cat, W_out, delta, ln_gamma.reshape(1, D),
                ln_beta.reshape(1, D), fnl_embeds, h2, e_bar)


# speedup vs baseline: 4.2431x; 1.0216x over previous
import jax
import jax.numpy as jnp
from jax import lax
from jax.experimental import pallas as pl
from jax.experimental.pallas import tpu as pltpu
from jax.experimental.pallas import tpu_sc as plsc

N = 10000
D = 256
NHEAD = 8
HDIM = 32
LB = 64
NC = 2
NS = 16
RPT = N // NS
F32 = jnp.float32
I32 = jnp.int32

E_DRP = 40000
E_PK = 120000
EP_DRP = 40960
EP_PK = 122880
SCH = 320
ACH = 80
ANCH = (EP_DRP // NS) // ACH
ZB = 80

_SC_PARAMS = pltpu.CompilerParams(use_tc_tiling_on_sc=False,
                                  needs_layout_passes=False)


def _mesh():
    return plsc.VectorSubcoreMesh(core_axis_name="c", subcore_axis_name="s",
                                  num_cores=NC, num_subcores=NS)


def _splat(i):
    return jnp.full((16,), 0, I32) + i


def _make_spmm2(nb, ep, ch, write_y1=True, interpret=False):
    nb2 = nb // 2
    per_tile = ep // NS
    nch = per_tile // ch

    def body(row_h, col_h, val_h, x_h, y1_h, y2_h,
             idxr, idxc, vals, rows0, rows1, zbuf, accA, accB,
             gsem0, gsem1, ssem0, ssem1):
        c = lax.axis_index("c")
        s = lax.axis_index("s")
        rows = (rows0, rows1)
        gsem = (gsem0, gsem1)
        ssem = (ssem0, ssem1)
        pltpu.sync_copy(row_h.at[s], idxr)
        pltpu.sync_copy(col_h.at[s], idxc)
        pltpu.sync_copy(val_h.at[s], vals)

        @pl.loop(0, ZB)
        def _(i):
            for j in range(LB // 16):
                zbuf[i, pl.ds(j * 16, 16)] = jnp.zeros((16,), F32)

        def zero_buf(buf):
            for k in range(RPT // ZB):
                pltpu.sync_copy(zbuf, buf.at[pl.ds(s * RPT + k * ZB, ZB)])
            if RPT % ZB:
                pltpu.sync_copy(
                    zbuf.at[pl.ds(0, RPT % ZB)],
                    buf.at[pl.ds(s * RPT + (RPT // ZB) * ZB, RPT % ZB)])

        def edges(src, dst):
            def fire_g(t, p):
                pltpu.async_copy(src.at[idxc.at[t]], rows[p], gsem[p])

            def wait_g(t, p):
                pltpu.make_async_copy(src.at[idxc.at[t]], rows[p],
                                      gsem[p]).wait()

            def fire_s(t, p):
                pltpu.async_copy(rows[p], dst.at[idxr.at[t]], ssem[p],
                                 add=True)

            def wait_s(t, p):
                pltpu.make_async_copy(rows[p], dst.at[idxr.at[t]],
                                      ssem[p]).wait()

            def scale(p, t):
                buf = rows[p]

                @pl.loop(0, ch, unroll=8)
                def _(i):
                    v = plsc.load_gather(vals.at[t], [_splat(i)])
                    for j in range(LB // 16):
                        sl = pl.ds(j * 16, 16)
                        buf[i, sl] = buf[i, sl] * v

            fire_g(0, 0)
            wait_g(0, 0)
            fire_g(1, 1)
            scale(0, 0)
            fire_s(0, 0)

            @pl.loop(0, (nch - 2) // 2)
            def _(tt):
                t = 2 * tt + 1
                wait_g(t, 1)
                wait_s(t - 1, 0)
                fire_g(t + 1, 0)
                scale(1, t)
                fire_s(t, 1)
                wait_g(t + 1, 0)
                wait_s(t, 1)
                fire_g(t + 2, 1)
                scale(0, t + 1)
                fire_s(t + 1, 0)

            t_last = nch - 1
            wait_g(t_last, 1)
            wait_s(t_last - 1, 0)
            scale(1, t_last)
            fire_s(t_last, 1)
            wait_s(t_last, 1)

        @pl.loop(0, nb2)
        def _(b):
            blk = c * nb2 + b
            own = pl.ds(s * RPT, RPT)
            pltpu.sync_copy(x_h.at[blk, own], accA.at[own])
            zero_buf(accB)
            plsc.subcore_barrier()
            edges(accA, accB)
            plsc.subcore_barrier()
            if write_y1:
                pltpu.sync_copy(accB.at[own], y1_h.at[blk, own])
            zero_buf(accA)
            plsc.subcore_barrier()
            edges(accB, accA)
            plsc.subcore_barrier()
            pltpu.sync_copy(accA.at[own], y2_h.at[blk, own])
            plsc.subcore_barrier()

    out = (jax.ShapeDtypeStruct((nb, N, LB), F32),
           jax.ShapeDtypeStruct((nb, N, LB), F32))
    return pl.kernel(
        body,
        out_type=out,
        mesh=_mesh(),
        interpret=interpret,
        compiler_params=_SC_PARAMS,
        scratch_types=[
            pltpu.VMEM((nch, ch), I32),
            pltpu.VMEM((nch, ch), I32),
            pltpu.VMEM((nch, ch), F32),
            pltpu.VMEM((ch, LB), F32),
            pltpu.VMEM((ch, LB), F32),
            pltpu.VMEM((ZB, LB), F32),
            pltpu.VMEM_SHARED((N, LB), F32),
            pltpu.VMEM_SHARED((N, LB), F32),
            pltpu.SemaphoreType.DMA,
            pltpu.SemaphoreType.DMA,
            pltpu.SemaphoreType.DMA,
            pltpu.SemaphoreType.DMA,
        ],
    )


def _make_attn(interpret=False):
    per_tile = EP_DRP // NS
    scale = 1.0 / (HDIM ** 0.5)
    nch = ANCH
    NF = RPT // ACH
    NR = RPT - NF * ACH

    def body(row_h, col_h, qb_h, kb_h, vb_h, agg_h,
             idxr, idxc, rowsA0, rowsA1, rowsB0, rowsB1, vbuf0, vbuf1,
             exps, contrib, dbuf, denom, aggacc,
             sem0, sem1, sem2, sem3):
        c = lax.axis_index("c")
        s = lax.axis_index("s")
        rowsA = (rowsA0, rowsA1)
        rowsB = (rowsB0, rowsB1)
        vbuf = (vbuf0, vbuf1)
        qsem = (sem0, sem1)
        ksem = (sem2, sem3)
        pltpu.sync_copy(row_h.at[s], idxr)
        pltpu.sync_copy(col_h.at[s], idxc)

        @pl.loop(0, ACH)
        def _(i):
            contrib[i, pl.ds(0, 16)] = jnp.zeros((16,), F32)

        for k in range(NF):
            pltpu.sync_copy(contrib, denom.at[pl.ds(s * RPT + k * ACH, ACH)])
        if NR:
            pltpu.sync_copy(contrib.at[pl.ds(0, NR)],
                            denom.at[pl.ds(s * RPT + NF * ACH, NR)])
        plsc.subcore_barrier()

        lanes0 = lax.iota(I32, 16)

        def fire_qk(t, p):
            pltpu.async_copy(qb_h.at[c].at[idxr.at[t]], rowsA[p], qsem[p])
            pltpu.async_copy(kb_h.at[c].at[idxc.at[t]], rowsB[p], ksem[p])

        def wait_qk(t, p):
            pltpu.make_async_copy(qb_h.at[c].at[idxr.at[t]], rowsA[p],
                                  qsem[p]).wait()
            pltpu.make_async_copy(kb_h.at[c].at[idxc.at[t]], rowsB[p],
                                  ksem[p]).wait()

        def logits(t, p):
            bufA = rowsA[p]
            bufB = rowsB[p]

            @pl.loop(0, ACH // 16)
            def _(g):
                lanes = g * 16 + lanes0
                init = (jnp.zeros((16,), F32),) * 4

                @pl.loop(0, HDIM, init_carry=init, unroll=4)
                def accs(d, carry):
                    outs = []
                    for h in range(4):
                        colv = jnp.full((16,), h * HDIM, I32) + d
                        q = plsc.load_gather(bufA, [lanes, colv])
                        k = plsc.load_gather(bufB, [lanes, colv])
                        outs.append(carry[h] + q * k)
                    return tuple(outs)

                ge = s * per_tile + t * ACH + lanes
                valid = ge < E_DRP
                for h in range(4):
                    a = accs[h] * scale
                    a = jnp.where(a >= 0, a, 0.2 * a)
                    a = jnp.clip(a, -20.0, 20.0)
                    e = jnp.where(valid, jnp.exp(a), 0.0)
                    hv = jnp.full((16,), h, I32)
                    plsc.store_scatter(contrib, [lanes, hv], e)
                    plsc.store_scatter(exps, [t * ACH + lanes, hv], e)

            pltpu.sync_copy(contrib, denom.at[idxr.at[t]], add=True)

        fire_qk(0, 0)
        wait_qk(0, 0)
        fire_qk(1, 1)
        logits(0, 0)

        @pl.loop(0, (nch - 2) // 2)
        def _(tt):
            t = 2 * tt + 1
            wait_qk(t, 1)
            fire_qk(t + 1, 0)
            logits(t, 1)
            wait_qk(t + 1, 0)
            fire_qk(t + 2, 1)
            logits(t + 1, 0)

        wait_qk(nch - 1, 1)
        logits(nch - 1, 1)
        plsc.subcore_barrier()

        vsem = qsem
        ssem = ksem

        for j in range(2):
            vblk = 2 * c + j

            @pl.loop(0, ACH)
            def _(i):
                for j2 in range(LB // 16):
                    vbuf0[i, pl.ds(j2 * 16, 16)] = jnp.zeros((16,), F32)

            for k in range(NF):
                pltpu.sync_copy(vbuf0, aggacc.at[pl.ds(s * RPT + k * ACH, ACH)])
            if NR:
                pltpu.sync_copy(vbuf0.at[pl.ds(0, NR)],
                                aggacc.at[pl.ds(s * RPT + NF * ACH, NR)])
            plsc.subcore_barrier()

            def fire_v(t, p):
                pltpu.async_copy(vb_h.at[vblk].at[idxc.at[t]], vbuf[p],
                                 vsem[p])

            def wait_v(t, p):
                pltpu.make_async_copy(vb_h.at[vblk].at[idxc.at[t]], vbuf[p],
                                      vsem[p]).wait()

            def fire_s(t, p):
                pltpu.async_copy(vbuf[p], aggacc.at[idxr.at[t]], ssem[p],
                                 add=True)

            def wait_s(t, p):
                pltpu.make_async_copy(vbuf[p], aggacc.at[idxr.at[t]],
                                      ssem[p]).wait()

            def vscale(t, p):
                buf = vbuf[p]

                @pl.loop(0, ACH, unroll=4)
                def _(i):
                    le = _splat(t * ACH + i)
                    for j2 in range(LB // 16):
                        hv = jnp.full((16,), j2 // 2, I32)
                        w = plsc.load_gather(exps, [le, hv + (2 * j)])
                        sl = pl.ds(j2 * 16, 16)
                        buf[i, sl] = buf[i, sl] * w

            fire_v(0, 0)
            wait_v(0, 0)
            fire_v(1, 1)
            vscale(0, 0)
            fire_s(0, 0)

            @pl.loop(0, (nch - 2) // 2)
            def _(tt):
                t = 2 * tt + 1
                wait_v(t, 1)
                wait_s(t - 1, 0)
                fire_v(t + 1, 0)
                vscale(t, 1)
                fire_s(t, 1)
                wait_v(t + 1, 0)
                wait_s(t, 1)
                fire_v(t + 2, 1)
                vscale(t + 1, 0)
                fire_s(t + 1, 0)

            wait_v(nch - 1, 1)
            wait_s(nch - 2, 0)
            vscale(nch - 1, 1)
            fire_s(nch - 1, 1)
            wait_s(nch - 1, 1)
            plsc.subcore_barrier()

            for k in range(NF + (1 if NR else 0)):
                nrows = ACH if k < NF else NR
                base = s * RPT + k * ACH
                if nrows == ACH:
                    pltpu.sync_copy(aggacc.at[pl.ds(base, ACH)], vbuf0)
                    pltpu.sync_copy(denom.at[pl.ds(base, ACH)], dbuf)
                else:
                    pltpu.sync_copy(aggacc.at[pl.ds(base, nrows)],
                                    vbuf0.at[pl.ds(0, nrows)])
                    pltpu.sync_copy(denom.at[pl.ds(base, nrows)],
                                    dbuf.at[pl.ds(0, nrows)])

                @pl.loop(0, nrows, unroll=2)
                def _(i):
                    d = dbuf[i, pl.ds(0, 16)]
                    dbuf[i, pl.ds(0, 16)] = 1.0 / (d + 1e-10)
                    iv = _splat(i)
                    for j2 in range(LB // 16):
                        hv = jnp.full((16,), j2 // 2, I32)
                        r = plsc.load_gather(dbuf, [iv, hv + (2 * j)])
                        sl = pl.ds(j2 * 16, 16)
                        vbuf0[i, sl] = vbuf0[i, sl] * r

                if nrows == ACH:
                    pltpu.sync_copy(vbuf0, agg_h.at[vblk, pl.ds(base, ACH)])
                else:
                    pltpu.sync_copy(vbuf0.at[pl.ds(0, nrows)],
                                    agg_h.at[vblk, pl.ds(base, nrows)])
            plsc.subcore_barrier()

    out = jax.ShapeDtypeStruct((2 * NC, N, LB), F32)
    return pl.kernel(
        body,
        out_type=out,
        mesh=_mesh(),
        interpret=interpret,
        compiler_params=_SC_PARAMS,
        scratch_types=[
            pltpu.VMEM((ANCH, ACH), I32),
            pltpu.VMEM((ANCH, ACH), I32),
            pltpu.VMEM((ACH, 2 * LB), F32),
            pltpu.VMEM((ACH, 2 * LB), F32),
            pltpu.VMEM((ACH, 2 * LB), F32),
            pltpu.VMEM((ACH, 2 * LB), F32),
            pltpu.VMEM((ACH, LB), F32),
            pltpu.VMEM((ACH, LB), F32),
            pltpu.VMEM((per_tile, 4), F32),
            pltpu.VMEM((ACH, 16), F32),
            pltpu.VMEM((ACH, 16), F32),
            pltpu.VMEM_SHARED((N, 16), F32),
            pltpu.VMEM_SHARED((N, LB), F32),
            pltpu.SemaphoreType.DMA,
            pltpu.SemaphoreType.DMA,
            pltpu.SemaphoreType.DMA,
            pltpu.SemaphoreType.DMA,
        ],
    )


ROWB = 1000
G = N // ROWB


def _row_spec(width):
    return pl.BlockSpec((ROWB, width), lambda i: (i, 0))


def _full_spec(h, w):
    return pl.BlockSpec((h, w), lambda i: (0, 0))


def _wd_body(fnl, wr, o):
    o[...] = fnl[...] * wr[...]


_wd = pl.pallas_call(
    _wd_body,
    grid=(G,),
    in_specs=[_row_spec(D), pl.BlockSpec((ROWB, 1), lambda i: (i, 0))],
    out_specs=_row_spec(D),
    out_shape=jax.ShapeDtypeStruct((N, D), F32),
)


def _add3_body(a, b, c, o):
    o[...] = a[...] + b[...] + c[...]


_add3 = pl.pallas_call(
    _add3_body,
    grid=(G,),
    in_specs=[_row_spec(D)] * 3,
    out_specs=_row_spec(D),
    out_shape=jax.ShapeDtypeStruct((N, D), F32),
)


def _qkv_body(wd2, eo, w, qkv, delta):
    d = eo[...] - 0.1 * wd2[...]
    delta[...] = d
    qkv[...] = jnp.dot(d, w[...], preferred_element_type=F32)


_qkv = pl.pallas_call(
    _qkv_body,
    grid=(G,),
    in_specs=[_row_spec(D), _row_spec(D), _full_spec(D, 3 * D)],
    out_specs=[_row_spec(3 * D), _row_spec(D)],
    out_shape=(jax.ShapeDtypeStruct((N, 3 * D), F32),
               jax.ShapeDtypeStruct((N, D), F32)),
)


def _fin_body(agg, wout, delta, g, bta, fnl, h2, eb, o):
    ao = jnp.dot(agg[...], wout[...], preferred_element_type=F32)
    x = delta[...] + ao
    mu = jnp.mean(x, axis=-1, keepdims=True)
    var = jnp.mean((x - mu) ** 2, axis=-1, keepdims=True)
    x = (x - mu) / jnp.sqrt(var + 1e-5) * g[...] + bta[...]
    x = jnp.where(x >= 0, x, 0.1 * x)
    x = jnp.where(x >= 0, x, 0.1 * x)
    o[...] = fnl[...] + x + h2[...] - eb[...]


_fin = pl.pallas_call(
    _fin_body,
    grid=(G,),
    in_specs=[_row_spec(D), _full_spec(D, D), _row_spec(D),
              _full_spec(1, D), _full_spec(1, D),
              _row_spec(D), _row_spec(D), _row_spec(D)],
    out_specs=_row_spec(D),
    out_shape=jax.ShapeDtypeStruct((N, D), F32),
)


_spmm_drp = _make_spmm2(12, EP_DRP, 256)
_spmm_pk = _make_spmm2(8, EP_PK, 160, write_y1=False)
_attn = _make_attn()


def _prep_edges(ei, vals, ep, ch):
    e = ei.shape[1]
    pad = ep - e
    row = jnp.concatenate([ei[0], jnp.zeros((pad,), I32)]).reshape(NS, -1, ch)
    col = jnp.concatenate([ei[1], jnp.zeros((pad,), I32)]).reshape(NS, -1, ch)
    v = jnp.concatenate([vals, jnp.zeros((pad,), F32)]).reshape(NS, -1, ch)
    return row, col, v


def _blocks4(x):
    return (x[:, :LB], x[:, LB:2 * LB], x[:, 2 * LB:3 * LB], x[:, 3 * LB:])


def _cat4(blocks):
    return jnp.concatenate(blocks, -1)


def kernel(ori_edge_index, ori_values, pk_edge_index, pk_values, mask,
           drp_edge_index, drp_values, edge_embeds1, withdraw_rate,
           W_qkv, W_out, ln_gamma, ln_beta, ini_embeds, fnl_embeds):
    drow, dcol, dval = _prep_edges(drp_edge_index, drp_values, EP_DRP, 256)
    prow, pcol, pval = _prep_edges(pk_edge_index, pk_values, EP_PK, 160)
    arow = drow.reshape(NS, ANCH, ACH)
    acol = dcol.reshape(NS, ANCH, ACH)

    wd0 = _wd(fnl_embeds, withdraw_rate)
    x0 = jnp.stack([*_blocks4(edge_embeds1), *_blocks4(wd0),
                    *_blocks4(ini_embeds)])
    y1, y2 = _spmm_drp(drow, dcol, dval, x0)
    lat1 = _cat4([y1[i] for i in range(4)])
    lat2 = _cat4([y2[i] for i in range(4)])
    wd2 = _cat4([y2[i] for i in range(4, 8)])
    h2 = _cat4([y2[i] for i in range(8, 12)])

    edge_embed = _add3(edge_embeds1, lat1, lat2)
    z0 = jnp.stack([*_blocks4(edge_embed), *_blocks4(ini_embeds)])
    _, z2 = _spmm_pk(prow, pcol, pval, z0)
    edge_out = _cat4([z2[i] for i in range(4)])
    e_bar = _cat4([z2[i] for i in range(4, 8)])

    qkv, delta = _qkv(wd2, edge_out, W_qkv)
    qb = jnp.stack([qkv[:, :128], qkv[:, 128:256]])
    kb = jnp.stack([qkv[:, 256:384], qkv[:, 384:512]])
    vb = jnp.stack([qkv[:, 512 + i * LB:512 + (i + 1) * LB] for i in range(4)])
    agg = _attn(arow, acol, qb, kb, vb)
    agg_cat = _cat4([agg[i] for i in range(4)])

    return _fin(agg_cat, W_out, delta, ln_gamma.reshape(1, D),
                ln_beta.reshape(1, D), fnl_embeds, h2, e_bar)
